# baseline (speedup 1.0000x reference)
import functools

import jax
import jax.numpy as jnp
import numpy as np
from jax import lax
from jax.experimental import pallas as pl
from jax.experimental.pallas import tpu as pltpu
from jax.experimental.pallas import tpu_sc as plsc

VOCAB = 100000
EMBED = 128
NUM_DENSE = 13
NUM_SPARSE = 26
BATCH = 4096
NFEAT = NUM_SPARSE + 1
NPAD = 32
_BOT_SLOT = 26

_NC = 2
_NS = 16
_NW = _NC * _NS
_CH = 512

_SPLITS = (2048, 2048)
_BB = 256


@functools.lru_cache(maxsize=None)
def _make_sc_gather(nidx):
    b_per_w = nidx // _NW
    nch = b_per_w // _CH
    assert b_per_w % _CH == 0

    @functools.partial(
        pl.kernel,
        mesh=plsc.VectorSubcoreMesh(core_axis_name="c", subcore_axis_name="s"),
        out_type=jax.ShapeDtypeStruct((nidx, EMBED), jnp.float32),
        scratch_types=[
            pltpu.VMEM((_CH,), jnp.int32),
            pltpu.VMEM((_CH, EMBED), jnp.float32),
            pltpu.SemaphoreType.DMA,
        ],
    )
    def sc_gather(table_hbm, idx_hbm, out_hbm, idx_v, rows_v, sem):
        wid = lax.axis_index("s") * _NC + lax.axis_index("c")
        base = wid * b_per_w
        for c in range(nch):
            off = base + c * _CH
            pltpu.sync_copy(idx_hbm.at[pl.ds(off, _CH)], idx_v)
            pltpu.async_copy(table_hbm.at[idx_v], rows_v, sem).wait()
            pltpu.sync_copy(rows_v, out_hbm.at[pl.ds(off, _CH)])

    return sc_gather


def _mm(a, b):
    return jnp.dot(a.astype(jnp.bfloat16), b,
                   preferred_element_type=jnp.float32)


def _tc_body(dense_ref, emb_ref, wb0, bb0, wb1, bb1, wb2, bb2,
             w0a, u_fold, bt0, wt1, bt1, wt2, bt2, wt3, bt3, wt4, bt4,
             out_ref):
    f32 = jnp.float32
    h = dense_ref[...]
    h = jnp.maximum(_mm(h, wb0[...]) + bb0[...], 0.0)
    h = jnp.maximum(_mm(h, wb1[...]) + bb1[...], 0.0)
    bot = jnp.maximum(_mm(h, wb2[...]) + bb2[...], 0.0)

    emb3 = emb_ref[...].reshape(_BB, NPAD, EMBED)
    slot = lax.broadcasted_iota(jnp.int32, (_BB, NPAD, EMBED), 1)
    feat3 = jnp.where(slot == _BOT_SLOT, bot.reshape(_BB, 1, EMBED), emb3)
    feat3 = feat3.astype(jnp.bfloat16)
    xact = lax.dot_general(
        feat3, feat3, (((2,), (2,)), ((0,), (0,))), preferred_element_type=f32
    )
    xf = xact.reshape(_BB, NPAD * NPAD)

    t = _mm(bot, w0a[...]) + _mm(xf, u_fold[...]) + bt0[...]
    t = jnp.maximum(t, 0.0)
    t = jnp.maximum(_mm(t, wt1[...]) + bt1[...], 0.0)
    t = jnp.maximum(_mm(t, wt2[...]) + bt2[...], 0.0)
    t = jnp.maximum(_mm(t, wt3[...]) + bt3[...], 0.0)
    out_ref[...] = _mm(t, wt4[...]) + bt4[...]


def _full(shape):
    return pl.BlockSpec(shape, lambda i: (0, 0))


def _tc_forward(dense, emb, wb0, bb0, wb1, bb1, wb2, bb2,
                w0a, u_fold, bt0, wt1, bt1, wt2, bt2, wt3, bt3, wt4, bt4):
    nb = dense.shape[0]
    specs = [
        pl.BlockSpec((_BB, NUM_DENSE), lambda i: (i, 0)),
        pl.BlockSpec((_BB * NPAD, EMBED), lambda i: (i, 0)),
        _full(wb0.shape), _full(bb0.shape),
        _full(wb1.shape), _full(bb1.shape),
        _full(wb2.shape), _full(bb2.shape),
        _full(w0a.shape), _full(u_fold.shape), _full(bt0.shape),
        _full(wt1.shape), _full(bt1.shape),
        _full(wt2.shape), _full(bt2.shape),
        _full(wt3.shape), _full(bt3.shape),
        _full(wt4.shape), _full(bt4.shape),
    ]
    return pl.pallas_call(
        _tc_body,
        grid=(nb // _BB,),
        in_specs=specs,
        out_specs=pl.BlockSpec((_BB, 1), lambda i: (i, 0)),
        out_shape=jax.ShapeDtypeStruct((nb, 1), jnp.float32),
    )(dense, emb, wb0, bb0, wb1, bb1, wb2, bb2,
      w0a, u_fold, bt0, wt1, bt1, wt2, bt2, wt3, bt3, wt4, bt4)


_M = np.where(np.arange(NFEAT) == 0, _BOT_SLOT, np.arange(NFEAT) - 1)
_IU, _JU = np.triu_indices(NFEAT)
_TRI_POS = np.asarray(_M[_IU] * NPAD + _M[_JU], dtype=np.int32)


def kernel(x, train, embedding_table, W_bot0, b_bot0, W_bot1, b_bot1,
           W_bot2, b_bot2, W_top0, b_top0, W_top1, b_top1, W_top2, b_top2,
           W_top3, b_top3, W_top4, b_top4):
    del train
    idx2 = x[:, NUM_DENSE:].astype(jnp.int32) % VOCAB
    idx = jnp.concatenate(
        [idx2, jnp.broadcast_to(idx2[:, :1], (BATCH, NPAD - NUM_SPARSE))],
        axis=1).reshape(-1)

    bf16 = jnp.bfloat16
    w0a = W_top0[:EMBED].astype(bf16)
    w0b = W_top0[EMBED:].astype(bf16)
    u_fold = jnp.zeros((NPAD * NPAD, W_top0.shape[1]), bf16)
    u_fold = u_fold.at[_TRI_POS].set(w0b)
    W_bot0, W_bot1, W_bot2, W_top1, W_top2, W_top3, W_top4 = (
        w.astype(bf16)
        for w in (W_bot0, W_bot1, W_bot2, W_top1, W_top2, W_top3, W_top4))

    row = lambda b: b.reshape(1, -1)
    starts = np.concatenate([[0], np.cumsum(_SPLITS)])
    embs = [
        _make_sc_gather(n * NPAD)(
            embedding_table,
            lax.slice_in_dim(idx, int(starts[h]) * NPAD,
                             int(starts[h + 1]) * NPAD))
        for h, n in enumerate(_SPLITS)
    ]
    outs = []
    for h, n in enumerate(_SPLITS):
        dense_h = lax.slice_in_dim(x, int(starts[h]), int(starts[h + 1]))[:, :NUM_DENSE]
        outs.append(_tc_forward(
            dense_h, embs[h], W_bot0, row(b_bot0), W_bot1, row(b_bot1),
            W_bot2, row(b_bot2), w0a, u_fold, row(b_top0), W_top1,
            row(b_top1), W_top2, row(b_top2), W_top3, row(b_top3), W_top4,
            row(b_top4)))
    return jnp.concatenate(outs, axis=0)


# speedup vs baseline: 1.0106x; 1.0106x over previous
import functools

import jax
import jax.numpy as jnp
import numpy as np
from jax import lax
from jax.experimental import pallas as pl
from jax.experimental.pallas import tpu as pltpu
from jax.experimental.pallas import tpu_sc as plsc

VOCAB = 100000
EMBED = 128
NUM_DENSE = 13
NUM_SPARSE = 26
BATCH = 4096
NFEAT = NUM_SPARSE + 1
NPAD = 32
_BOT_SLOT = 26

_NC = 2
_NS = 16
_NW = _NC * _NS
_CH = 256
_NBUF = 3

_SPLITS = (2048, 2048)
_BB = 256


@functools.lru_cache(maxsize=None)
def _make_sc_gather(nidx):
    b_per_w = nidx // _NW
    nch = b_per_w // _CH
    assert b_per_w % _CH == 0

    @functools.partial(
        pl.kernel,
        mesh=plsc.VectorSubcoreMesh(core_axis_name="c", subcore_axis_name="s"),
        out_type=jax.ShapeDtypeStruct((nidx, EMBED), jnp.float32),
        scratch_types=[
            pltpu.VMEM((b_per_w,), jnp.int32),
            pltpu.VMEM((_NBUF, _CH, EMBED), jnp.float32),
            pltpu.SemaphoreType.DMA((_NBUF,)),
            pltpu.SemaphoreType.DMA((_NBUF,)),
        ],
    )
    def sc_gather(table_hbm, idx_hbm, out_hbm, idx_v, rows_v, gsem, osem):
        wid = lax.axis_index("s") * _NC + lax.axis_index("c")
        base = wid * b_per_w
        pltpu.sync_copy(idx_hbm.at[pl.ds(base, b_per_w)], idx_v)
        gat = [None] * nch
        out = [None] * nch
        for c in range(nch):
            k = c % _NBUF
            if c >= _NBUF:
                out[c - _NBUF].wait()
            gat[c] = pltpu.async_copy(
                table_hbm.at[idx_v.at[pl.ds(c * _CH, _CH)]],
                rows_v.at[k], gsem.at[k])
            if c >= 1:
                p = (c - 1) % _NBUF
                gat[c - 1].wait()
                out[c - 1] = pltpu.async_copy(
                    rows_v.at[p], out_hbm.at[pl.ds(base + (c - 1) * _CH, _CH)],
                    osem.at[p])
        gat[nch - 1].wait()
        out[nch - 1] = pltpu.async_copy(
            rows_v.at[(nch - 1) % _NBUF],
            out_hbm.at[pl.ds(base + (nch - 1) * _CH, _CH)],
            osem.at[(nch - 1) % _NBUF])
        for c in range(max(0, nch - _NBUF), nch):
            out[c].wait()

    return sc_gather


def _mm(a, b):
    return jnp.dot(a.astype(jnp.bfloat16), b,
                   preferred_element_type=jnp.float32)


def _tc_body(dense_ref, emb_ref, wb0, bb0, wb1, bb1, wb2, bb2,
             w0a, u_fold, bt0, wt1, bt1, wt2, bt2, wt3, bt3, wt4, bt4,
             out_ref):
    f32 = jnp.float32
    h = dense_ref[...]
    h = jnp.maximum(_mm(h, wb0[...]) + bb0[...], 0.0)
    h = jnp.maximum(_mm(h, wb1[...]) + bb1[...], 0.0)
    bot = jnp.maximum(_mm(h, wb2[...]) + bb2[...], 0.0)

    emb3 = emb_ref[...].reshape(_BB, NPAD, EMBED)
    slot = lax.broadcasted_iota(jnp.int32, (_BB, NPAD, EMBED), 1)
    feat3 = jnp.where(slot == _BOT_SLOT, bot.reshape(_BB, 1, EMBED), emb3)
    feat3 = feat3.astype(jnp.bfloat16)
    xact = lax.dot_general(
        feat3, feat3, (((2,), (2,)), ((0,), (0,))), preferred_element_type=f32
    )
    xf = xact.reshape(_BB, NPAD * NPAD)

    t = _mm(bot, w0a[...]) + _mm(xf, u_fold[...]) + bt0[...]
    t = jnp.maximum(t, 0.0)
    t = jnp.maximum(_mm(t, wt1[...]) + bt1[...], 0.0)
    t = jnp.maximum(_mm(t, wt2[...]) + bt2[...], 0.0)
    t = jnp.maximum(_mm(t, wt3[...]) + bt3[...], 0.0)
    out_ref[...] = _mm(t, wt4[...]) + bt4[...]


def _full(shape):
    return pl.BlockSpec(shape, lambda i: (0, 0))


def _tc_forward(dense, emb, wb0, bb0, wb1, bb1, wb2, bb2,
                w0a, u_fold, bt0, wt1, bt1, wt2, bt2, wt3, bt3, wt4, bt4):
    nb = dense.shape[0]
    specs = [
        pl.BlockSpec((_BB, NUM_DENSE), lambda i: (i, 0)),
        pl.BlockSpec((_BB * NPAD, EMBED), lambda i: (i, 0)),
        _full(wb0.shape), _full(bb0.shape),
        _full(wb1.shape), _full(bb1.shape),
        _full(wb2.shape), _full(bb2.shape),
        _full(w0a.shape), _full(u_fold.shape), _full(bt0.shape),
        _full(wt1.shape), _full(bt1.shape),
        _full(wt2.shape), _full(bt2.shape),
        _full(wt3.shape), _full(bt3.shape),
        _full(wt4.shape), _full(bt4.shape),
    ]
    return pl.pallas_call(
        _tc_body,
        grid=(nb // _BB,),
        in_specs=specs,
        out_specs=pl.BlockSpec((_BB, 1), lambda i: (i, 0)),
        out_shape=jax.ShapeDtypeStruct((nb, 1), jnp.float32),
    )(dense, emb, wb0, bb0, wb1, bb1, wb2, bb2,
      w0a, u_fold, bt0, wt1, bt1, wt2, bt2, wt3, bt3, wt4, bt4)


_M = np.where(np.arange(NFEAT) == 0, _BOT_SLOT, np.arange(NFEAT) - 1)
_IU, _JU = np.triu_indices(NFEAT)
_TRI_POS = np.asarray(_M[_IU] * NPAD + _M[_JU], dtype=np.int32)


def kernel(x, train, embedding_table, W_bot0, b_bot0, W_bot1, b_bot1,
           W_bot2, b_bot2, W_top0, b_top0, W_top1, b_top1, W_top2, b_top2,
           W_top3, b_top3, W_top4, b_top4):
    del train
    idx2 = x[:, NUM_DENSE:].astype(jnp.int32) % VOCAB
    idx = jnp.concatenate(
        [idx2, jnp.broadcast_to(idx2[:, :1], (BATCH, NPAD - NUM_SPARSE))],
        axis=1).reshape(-1)

    bf16 = jnp.bfloat16
    w0a = W_top0[:EMBED].astype(bf16)
    w0b = W_top0[EMBED:].astype(bf16)
    u_fold = jnp.zeros((NPAD * NPAD, W_top0.shape[1]), bf16)
    u_fold = u_fold.at[_TRI_POS].set(w0b)
    W_bot0, W_bot1, W_bot2, W_top1, W_top2, W_top3, W_top4 = (
        w.astype(bf16)
        for w in (W_bot0, W_bot1, W_bot2, W_top1, W_top2, W_top3, W_top4))

    row = lambda b: b.reshape(1, -1)
    starts = np.concatenate([[0], np.cumsum(_SPLITS)])
    embs = [
        _make_sc_gather(n * NPAD)(
            embedding_table,
            lax.slice_in_dim(idx, int(starts[h]) * NPAD,
                             int(starts[h + 1]) * NPAD))
        for h, n in enumerate(_SPLITS)
    ]
    outs = []
    for h, n in enumerate(_SPLITS):
        dense_h = lax.slice_in_dim(x, int(starts[h]), int(starts[h + 1]))[:, :NUM_DENSE]
        outs.append(_tc_forward(
            dense_h, embs[h], W_bot0, row(b_bot0), W_bot1, row(b_bot1),
            W_bot2, row(b_bot2), w0a, u_fold, row(b_top0), W_top1,
            row(b_top1), W_top2, row(b_top2), W_top3, row(b_top3), W_top4,
            row(b_top4)))
    return jnp.concatenate(outs, axis=0)


# speedup vs baseline: 1.0663x; 1.0551x over previous
import functools

import jax
import jax.numpy as jnp
import numpy as np
from jax import lax
from jax.experimental import pallas as pl
from jax.experimental.pallas import tpu as pltpu
from jax.experimental.pallas import tpu_sc as plsc

VOCAB = 100000
EMBED = 128
NUM_DENSE = 13
NUM_SPARSE = 26
BATCH = 4096
NFEAT = NUM_SPARSE + 1
NPAD = 32
_BOT_SLOT = 26

_NC = 2
_NS = 16
_NW = _NC * _NS
_CH = 256
_NBUF = 3

_SPLITS = (2048, 2048)
_BB = 512


@functools.lru_cache(maxsize=None)
def _make_sc_gather(nidx):
    b_per_w = nidx // _NW
    nch = b_per_w // _CH
    assert b_per_w % _CH == 0

    @functools.partial(
        pl.kernel,
        mesh=plsc.VectorSubcoreMesh(core_axis_name="c", subcore_axis_name="s"),
        out_type=jax.ShapeDtypeStruct((nidx, EMBED), jnp.float32),
        scratch_types=[
            pltpu.VMEM((b_per_w,), jnp.int32),
            pltpu.VMEM((_NBUF, _CH, EMBED), jnp.float32),
            pltpu.SemaphoreType.DMA((_NBUF,)),
            pltpu.SemaphoreType.DMA((_NBUF,)),
        ],
    )
    def sc_gather(table_hbm, idx_hbm, out_hbm, idx_v, rows_v, gsem, osem):
        wid = lax.axis_index("s") * _NC + lax.axis_index("c")
        base = wid * b_per_w
        pltpu.sync_copy(idx_hbm.at[pl.ds(base, b_per_w)], idx_v)
        gat = [None] * nch
        out = [None] * nch
        for c in range(nch):
            k = c % _NBUF
            if c >= _NBUF:
                out[c - _NBUF].wait()
            gat[c] = pltpu.async_copy(
                table_hbm.at[idx_v.at[pl.ds(c * _CH, _CH)]],
                rows_v.at[k], gsem.at[k])
            if c >= 1:
                p = (c - 1) % _NBUF
                gat[c - 1].wait()
                out[c - 1] = pltpu.async_copy(
                    rows_v.at[p], out_hbm.at[pl.ds(base + (c - 1) * _CH, _CH)],
                    osem.at[p])
        gat[nch - 1].wait()
        out[nch - 1] = pltpu.async_copy(
            rows_v.at[(nch - 1) % _NBUF],
            out_hbm.at[pl.ds(base + (nch - 1) * _CH, _CH)],
            osem.at[(nch - 1) % _NBUF])
        for c in range(max(0, nch - _NBUF), nch):
            out[c].wait()

    return sc_gather


def _mm(a, b):
    return jnp.dot(a.astype(jnp.bfloat16), b,
                   preferred_element_type=jnp.float32)


def _tc_body(dense_ref, emb_ref, wb0, bb0, wb1, bb1, wb2, bb2,
             w0a, u_fold, bt0, wt1, bt1, wt2, bt2, wt3, bt3, wt4, bt4,
             out_ref):
    f32 = jnp.float32
    h = dense_ref[...]
    h = jnp.maximum(_mm(h, wb0[...]) + bb0[...], 0.0)
    h = jnp.maximum(_mm(h, wb1[...]) + bb1[...], 0.0)
    bot = jnp.maximum(_mm(h, wb2[...]) + bb2[...], 0.0)

    emb3 = emb_ref[...].reshape(_BB, NPAD, EMBED)
    slot = lax.broadcasted_iota(jnp.int32, (_BB, NPAD, EMBED), 1)
    feat3 = jnp.where(slot == _BOT_SLOT, bot.reshape(_BB, 1, EMBED), emb3)
    feat3 = feat3.astype(jnp.bfloat16)
    xact = lax.dot_general(
        feat3, feat3, (((2,), (2,)), ((0,), (0,))), preferred_element_type=f32
    )
    xf = xact.reshape(_BB, NPAD * NPAD)

    t = _mm(bot, w0a[...]) + _mm(xf, u_fold[...]) + bt0[...]
    t = jnp.maximum(t, 0.0)
    t = jnp.maximum(_mm(t, wt1[...]) + bt1[...], 0.0)
    t = jnp.maximum(_mm(t, wt2[...]) + bt2[...], 0.0)
    t = jnp.maximum(_mm(t, wt3[...]) + bt3[...], 0.0)
    out_ref[...] = _mm(t, wt4[...]) + bt4[...]


def _full(shape):
    return pl.BlockSpec(shape, lambda i: (0, 0))


def _tc_forward(dense, emb, wb0, bb0, wb1, bb1, wb2, bb2,
                w0a, u_fold, bt0, wt1, bt1, wt2, bt2, wt3, bt3, wt4, bt4):
    nb = dense.shape[0]
    specs = [
        pl.BlockSpec((_BB, NUM_DENSE), lambda i: (i, 0)),
        pl.BlockSpec((_BB * NPAD, EMBED), lambda i: (i, 0)),
        _full(wb0.shape), _full(bb0.shape),
        _full(wb1.shape), _full(bb1.shape),
        _full(wb2.shape), _full(bb2.shape),
        _full(w0a.shape), _full(u_fold.shape), _full(bt0.shape),
        _full(wt1.shape), _full(bt1.shape),
        _full(wt2.shape), _full(bt2.shape),
        _full(wt3.shape), _full(bt3.shape),
        _full(wt4.shape), _full(bt4.shape),
    ]
    return pl.pallas_call(
        _tc_body,
        grid=(nb // _BB,),
        in_specs=specs,
        out_specs=pl.BlockSpec((_BB, 1), lambda i: (i, 0)),
        out_shape=jax.ShapeDtypeStruct((nb, 1), jnp.float32),
    )(dense, emb, wb0, bb0, wb1, bb1, wb2, bb2,
      w0a, u_fold, bt0, wt1, bt1, wt2, bt2, wt3, bt3, wt4, bt4)


_M = np.where(np.arange(NFEAT) == 0, _BOT_SLOT, np.arange(NFEAT) - 1)
_IU, _JU = np.triu_indices(NFEAT)
_TRI_POS = np.asarray(_M[_IU] * NPAD + _M[_JU], dtype=np.int32)


def kernel(x, train, embedding_table, W_bot0, b_bot0, W_bot1, b_bot1,
           W_bot2, b_bot2, W_top0, b_top0, W_top1, b_top1, W_top2, b_top2,
           W_top3, b_top3, W_top4, b_top4):
    del train
    idx2 = x[:, NUM_DENSE:].astype(jnp.int32) % VOCAB
    idx = jnp.concatenate(
        [idx2, jnp.broadcast_to(idx2[:, :1], (BATCH, NPAD - NUM_SPARSE))],
        axis=1).reshape(-1)

    bf16 = jnp.bfloat16
    w0a = W_top0[:EMBED].astype(bf16)
    w0b = W_top0[EMBED:].astype(bf16)
    u_fold = jnp.zeros((NPAD * NPAD, W_top0.shape[1]), bf16)
    u_fold = u_fold.at[_TRI_POS].set(w0b)
    W_bot0, W_bot1, W_bot2, W_top1, W_top2, W_top3, W_top4 = (
        w.astype(bf16)
        for w in (W_bot0, W_bot1, W_bot2, W_top1, W_top2, W_top3, W_top4))

    row = lambda b: b.reshape(1, -1)
    starts = np.concatenate([[0], np.cumsum(_SPLITS)])
    embs = [
        _make_sc_gather(n * NPAD)(
            embedding_table,
            lax.slice_in_dim(idx, int(starts[h]) * NPAD,
                             int(starts[h + 1]) * NPAD))
        for h, n in enumerate(_SPLITS)
    ]
    outs = []
    for h, n in enumerate(_SPLITS):
        dense_h = lax.slice_in_dim(x, int(starts[h]), int(starts[h + 1]))[:, :NUM_DENSE]
        outs.append(_tc_forward(
            dense_h, embs[h], W_bot0, row(b_bot0), W_bot1, row(b_bot1),
            W_bot2, row(b_bot2), w0a, u_fold, row(b_top0), W_top1,
            row(b_top1), W_top2, row(b_top2), W_top3, row(b_top3), W_top4,
            row(b_top4)))
    return jnp.concatenate(outs, axis=0)
